# baseline (speedup 1.0000x reference)
import functools

import jax
import jax.numpy as jnp
from jax import lax
from jax.experimental import pallas as pl
from jax.experimental.pallas import tpu as pltpu
from jax.experimental.pallas import tpu_sc as plsc

DIM = 128
LANES = 16
GROUPS = DIM // LANES
CHUNK = 128
NB = 2
NS = 2


@functools.partial(jax.jit, static_argnames=("rows",))
def _sc_permute(xf, perm_f, rows):
    info = plsc.get_sparse_core_info()
    nc, ns = info.num_cores, info.num_subcores
    nw = nc * ns
    rows_per_w = rows // nw
    n_chunks = rows_per_w // CHUNK
    cw = CHUNK * DIM
    assert n_chunks % NS == 0 and n_chunks >= 2 * NS

    mesh = plsc.VectorSubcoreMesh(core_axis_name="c", subcore_axis_name="s")

    @functools.partial(
        pl.kernel,
        out_type=jax.ShapeDtypeStruct((rows * DIM,), jnp.float32),
        mesh=mesh,
        scratch_types=[
            pltpu.VMEM((DIM,), jnp.float32),
            [pltpu.VMEM((cw,), jnp.float32) for _ in range(NB)],
            [pltpu.VMEM((cw,), jnp.float32) for _ in range(NB)],
            pltpu.VMEM_SHARED((ns, NS, cw), jnp.float32),
            [pltpu.SemaphoreType.DMA for _ in range(NB)],
            [pltpu.SemaphoreType.DMA for _ in range(NS)],
            [pltpu.SemaphoreType.DMA for _ in range(NS)],
        ],
        compiler_params=pltpu.CompilerParams(needs_layout_passes=False),
    )
    def body(x_hbm, p_hbm, o_hbm, p_v, in_v, ot_v, sh, s_in, s_x, s_o):
        sid = lax.axis_index("s")
        wid = sid * nc + lax.axis_index("c")
        pltpu.sync_copy(p_hbm, p_v)
        idx_g = [
            p_v[pl.ds(g * LANES, LANES)].astype(jnp.int32) for g in range(GROUPS)
        ]
        w_base = wid * rows_per_w * DIM

        def start_in(ci, b):
            pltpu.async_copy(x_hbm.at[pl.ds(w_base + ci * cw, cw)], in_v[b], s_in[b])

        def wait_in(b):
            pltpu.make_async_copy(x_hbm.at[pl.ds(0, cw)], in_v[b], s_in[b]).wait()

        def start_x(b, t):
            pltpu.async_copy(ot_v[b], sh.at[sid, t], s_x[t])

        def wait_x(t):
            pltpu.make_async_copy(ot_v[0], sh.at[sid, t], s_x[t]).wait()

        def start_o(ci, t):
            pltpu.async_copy(sh.at[sid, t], o_hbm.at[pl.ds(w_base + ci * cw, cw)], s_o[t])

        def wait_o(t):
            pltpu.make_async_copy(sh.at[sid, t], o_hbm.at[pl.ds(0, cw)], s_o[t]).wait()

        def compute(bi, bo):
            @plsc.parallel_loop(0, CHUNK, step=1, unroll=2)
            def _(r):
                rb = r * DIM
                in_row = in_v[bi].at[pl.ds(rb, DIM)]
                out_row = ot_v[bo].at[pl.ds(rb, DIM)]
                vals = [plsc.load_gather(in_row, [idx_g[g]]) for g in range(GROUPS)]
                for g in range(GROUPS):
                    out_row[pl.ds(g * LANES, LANES)] = vals[g]

        def step(ci, b, t, *, first_ring, prefetch, publish_slot):
            wait_in(b)
            compute(b, b)
            if not first_ring:
                wait_o(t)
            start_x(b, t)
            if prefetch:
                start_in(ci + NB, b)
            if publish_slot is not None:
                wait_x(publish_slot)
                start_o(ci - 1, publish_slot)

        for b in range(NB):
            start_in(b, b)
        for ci in range(NS):
            step(ci, ci % NB, ci % NS, first_ring=True, prefetch=True,
                 publish_slot=(ci - 1) % NS if ci >= 1 else None)

        def steady(k, _):
            for j in range(NS):
                step(NS * k + j, j % NB, j, first_ring=False, prefetch=True,
                     publish_slot=(j - 1) % NS)
            return 0

        lax.fori_loop(1, n_chunks // NS - 1, steady, 0)

        for j in range(NS):
            ci = n_chunks - NS + j
            step(ci, j % NB, j, first_ring=False,
                 prefetch=ci + NB < n_chunks, publish_slot=(j - 1) % NS)
        wait_x((n_chunks - 1) % NS)
        start_o(n_chunks - 1, (n_chunks - 1) % NS)
        for t in range(NS):
            wait_o(t)

    return body(xf, perm_f)


def kernel(x, permutation_matrix):
    b, h, s, d = x.shape
    rows = b * h * s
    xf = x.reshape(rows * d)
    out = _sc_permute(xf, permutation_matrix, rows)
    return out.reshape(x.shape)


# speedup vs baseline: 1.1060x; 1.1060x over previous
import functools

import jax
import jax.numpy as jnp
from jax import lax
from jax.experimental import pallas as pl
from jax.experimental.pallas import tpu as pltpu
from jax.experimental.pallas import tpu_sc as plsc

DIM = 128
LANES = 16
GROUPS = DIM // LANES
CHUNK = 128
NBI = 2
NBO = 4


@functools.partial(jax.jit, static_argnames=("rows",))
def _sc_permute(xf, perm_f, rows):
    info = plsc.get_sparse_core_info()
    nc, ns = info.num_cores, info.num_subcores
    nw = nc * ns
    rows_per_w = rows // nw
    n_chunks = rows_per_w // CHUNK
    cw = CHUNK * DIM
    assert n_chunks % NBO == 0 and n_chunks >= 2 * NBO

    mesh = plsc.VectorSubcoreMesh(core_axis_name="c", subcore_axis_name="s")

    @functools.partial(
        pl.kernel,
        out_type=jax.ShapeDtypeStruct((rows * DIM,), jnp.float32),
        mesh=mesh,
        scratch_types=[
            pltpu.VMEM((DIM,), jnp.float32),
            [pltpu.VMEM((cw,), jnp.float32) for _ in range(NBI)],
            [pltpu.VMEM((cw,), jnp.float32) for _ in range(NBO)],
            [pltpu.SemaphoreType.DMA for _ in range(NBI)],
            [pltpu.SemaphoreType.DMA for _ in range(NBO)],
        ],
        compiler_params=pltpu.CompilerParams(needs_layout_passes=False),
    )
    def body(x_hbm, p_hbm, o_hbm, p_v, in_v, out_v, s_in, s_out):
        wid = lax.axis_index("s") * nc + lax.axis_index("c")
        pltpu.sync_copy(p_hbm, p_v)
        idx_g = [
            p_v[pl.ds(g * LANES, LANES)].astype(jnp.int32) for g in range(GROUPS)
        ]
        w_base = wid * rows_per_w * DIM

        def start_in(ci, b):
            pltpu.async_copy(x_hbm.at[pl.ds(w_base + ci * cw, cw)], in_v[b], s_in[b])

        def wait_in(b):
            pltpu.make_async_copy(x_hbm.at[pl.ds(0, cw)], in_v[b], s_in[b]).wait()

        def start_out(ci, b):
            pltpu.async_copy(out_v[b], o_hbm.at[pl.ds(w_base + ci * cw, cw)], s_out[b])

        def wait_out(b):
            pltpu.make_async_copy(out_v[b], o_hbm.at[pl.ds(0, cw)], s_out[b]).wait()

        def compute(bi, bo):
            @plsc.parallel_loop(0, CHUNK, step=1, unroll=2)
            def _(r):
                rb = r * DIM
                in_row = in_v[bi].at[pl.ds(rb, DIM)]
                out_row = out_v[bo].at[pl.ds(rb, DIM)]
                vals = [plsc.load_gather(in_row, [idx_g[g]]) for g in range(GROUPS)]
                for g in range(GROUPS):
                    out_row[pl.ds(g * LANES, LANES)] = vals[g]

        for b in range(NBI):
            start_in(b, b)
        for ci in range(NBO):
            wait_in(ci % NBI)
            compute(ci % NBI, ci % NBO)
            start_out(ci, ci % NBO)
            start_in(ci + NBI, ci % NBI)

        def steady(k, _):
            for j in range(NBO):
                ci = NBO * k + j
                wait_in(j % NBI)
                wait_out(j)
                compute(j % NBI, j)
                start_out(ci, j)
                start_in(ci + NBI, j % NBI)
            return 0

        lax.fori_loop(1, n_chunks // NBO - 1, steady, 0)

        for j in range(NBO):
            ci = n_chunks - NBO + j
            wait_in(j % NBI)
            wait_out(j)
            compute(j % NBI, j)
            start_out(ci, j)
            if ci + NBI < n_chunks:
                start_in(ci + NBI, j % NBI)
        for j in range(NBO):
            wait_out(j)

    return body(xf, perm_f)


def kernel(x, permutation_matrix):
    b, h, s, d = x.shape
    rows = b * h * s
    xf = x.reshape(rows * d)
    out = _sc_permute(xf, permutation_matrix, rows)
    return out.reshape(x.shape)


# speedup vs baseline: 1.1276x; 1.0195x over previous
import functools

import jax
import jax.numpy as jnp
from jax import lax
from jax.experimental import pallas as pl
from jax.experimental.pallas import tpu as pltpu
from jax.experimental.pallas import tpu_sc as plsc

DIM = 128
LANES = 16
GROUPS = DIM // LANES
CHUNK = 64
NBI = 4
NBO = 4


@functools.partial(jax.jit, static_argnames=("rows",))
def _sc_permute(xf, perm_f, rows):
    info = plsc.get_sparse_core_info()
    nc, ns = info.num_cores, info.num_subcores
    nw = nc * ns
    rows_per_w = rows // nw
    n_chunks = rows_per_w // CHUNK
    cw = CHUNK * DIM
    assert n_chunks % NBO == 0 and n_chunks >= 2 * NBO

    mesh = plsc.VectorSubcoreMesh(core_axis_name="c", subcore_axis_name="s")

    @functools.partial(
        pl.kernel,
        out_type=jax.ShapeDtypeStruct((rows * DIM,), jnp.float32),
        mesh=mesh,
        scratch_types=[
            pltpu.VMEM((DIM,), jnp.float32),
            [pltpu.VMEM((cw,), jnp.float32) for _ in range(NBI)],
            [pltpu.VMEM((cw,), jnp.float32) for _ in range(NBO)],
            [pltpu.SemaphoreType.DMA for _ in range(NBI)],
            [pltpu.SemaphoreType.DMA for _ in range(NBO)],
        ],
        compiler_params=pltpu.CompilerParams(needs_layout_passes=False),
    )
    def body(x_hbm, p_hbm, o_hbm, p_v, in_v, out_v, s_in, s_out):
        wid = lax.axis_index("s") * nc + lax.axis_index("c")
        pltpu.sync_copy(p_hbm, p_v)
        idx_g = [
            p_v[pl.ds(g * LANES, LANES)].astype(jnp.int32) for g in range(GROUPS)
        ]
        w_base = wid * rows_per_w * DIM

        def start_in(ci, b):
            pltpu.async_copy(x_hbm.at[pl.ds(w_base + ci * cw, cw)], in_v[b], s_in[b])

        def wait_in(b):
            pltpu.make_async_copy(x_hbm.at[pl.ds(0, cw)], in_v[b], s_in[b]).wait()

        def start_out(ci, b):
            pltpu.async_copy(out_v[b], o_hbm.at[pl.ds(w_base + ci * cw, cw)], s_out[b])

        def wait_out(b):
            pltpu.make_async_copy(out_v[b], o_hbm.at[pl.ds(0, cw)], s_out[b]).wait()

        def compute(bi, bo):
            @plsc.parallel_loop(0, CHUNK, step=1, unroll=2)
            def _(r):
                rb = r * DIM
                in_row = in_v[bi].at[pl.ds(rb, DIM)]
                out_row = out_v[bo].at[pl.ds(rb, DIM)]
                vals = [plsc.load_gather(in_row, [idx_g[g]]) for g in range(GROUPS)]
                for g in range(GROUPS):
                    out_row[pl.ds(g * LANES, LANES)] = vals[g]

        for b in range(NBI):
            start_in(b, b)
        for ci in range(NBO):
            wait_in(ci % NBI)
            compute(ci % NBI, ci % NBO)
            start_out(ci, ci % NBO)
            start_in(ci + NBI, ci % NBI)

        def steady(k, _):
            for j in range(NBO):
                ci = NBO * k + j
                wait_in(j % NBI)
                wait_out(j)
                compute(j % NBI, j)
                start_out(ci, j)
                start_in(ci + NBI, j % NBI)
            return 0

        lax.fori_loop(1, n_chunks // NBO - 1, steady, 0)

        for j in range(NBO):
            ci = n_chunks - NBO + j
            wait_in(j % NBI)
            wait_out(j)
            compute(j % NBI, j)
            start_out(ci, j)
            if ci + NBI < n_chunks:
                start_in(ci + NBI, j % NBI)
        for j in range(NBO):
            wait_out(j)

    return body(xf, perm_f)


def kernel(x, permutation_matrix):
    b, h, s, d = x.shape
    rows = b * h * s
    xf = x.reshape(rows * d)
    out = _sc_permute(xf, permutation_matrix, rows)
    return out.reshape(x.shape)
